# Initial kernel scaffold; baseline (speedup 1.0000x reference)
import jax
import jax.numpy as jnp
from jax.experimental import pallas as pl


def kernel(coords, token_to_center_atom, token_exists_mask, residue_index, asym_id, token_bonds, is_ligand, W_pos, b_pos, W_edge, ln_gamma, ln_beta):
    raise NotImplementedError("write your pallas kernel here")


# speedup vs baseline: 2.0088x; 2.0088x over previous
import functools

import jax
import jax.numpy as jnp
from jax import lax
from jax.experimental import pallas as pl
from jax.experimental.pallas import tpu as pltpu
from jax.experimental.pallas import tpu_sc as plsc

_B = 4
_N = 2048
_K = 32
_NW = 32
_TPW = _B * _N // _NW
_RPW = _TPW
_EPW = _RPW * _K

_RB = 128
_NBLK = _N // _RB

_RE = 2048
_NEB = _B * _N * _K // _RE


def _sc_gather_x(coords_pad, t2c_flat):
    mesh = plsc.VectorSubcoreMesh(core_axis_name="c", subcore_axis_name="s")

    @functools.partial(
        pl.kernel,
        mesh=mesh,
        out_type=jax.ShapeDtypeStruct((_B * _N, 128), jnp.float32),
        scratch_types=[
            pltpu.VMEM((_TPW,), jnp.int32),
            pltpu.VMEM((_TPW, 128), jnp.float32),
            pltpu.SemaphoreType.DMA,
        ],
    )
    def k(coords_hbm, idx_hbm, out_hbm, idx_v, rows_v, sem):
        wid = lax.axis_index("s") * 2 + lax.axis_index("c")
        base = wid * _TPW
        pltpu.sync_copy(idx_hbm.at[pl.ds(base, _TPW)], idx_v)
        bb = (base // _N) * _N
        for c in range(_TPW // 16):
            sl = pl.ds(c * 16, 16)
            idx_v[sl] = idx_v[sl] + bb
        pltpu.async_copy(coords_hbm.at[idx_v], rows_v, sem).wait()
        pltpu.sync_copy(rows_v, out_hbm.at[pl.ds(base, _TPW)])

    return k(coords_pad, t2c_flat)


def _sc_edges(eidx_flat, tb2, res_flat, asym_flat, lig_flat):
    mesh = plsc.VectorSubcoreMesh(core_axis_name="c", subcore_axis_name="s")
    nedges = _B * _N * _K

    @functools.partial(
        pl.kernel,
        mesh=mesh,
        out_type=(
            jax.ShapeDtypeStruct((nedges,), jnp.float32),
            jax.ShapeDtypeStruct((nedges,), jnp.float32),
        ),
        compiler_params=pltpu.CompilerParams(needs_layout_passes=False),
        scratch_types=[
            pltpu.VMEM((_EPW,), jnp.int32),
            pltpu.VMEM((_B * _N,), jnp.int32),
            pltpu.VMEM((_B * _N,), jnp.int32),
            pltpu.VMEM((_B * _N,), jnp.int32),
            pltpu.VMEM((_N,), jnp.float32),
            pltpu.VMEM((_EPW,), jnp.float32),
            pltpu.VMEM((_EPW,), jnp.float32),
        ],
    )
    def k(eidx_hbm, tb_hbm, res_hbm, asym_hbm, lig_hbm, dpos_hbm, tbg_hbm,
          eidx_v, res_v, asym_v, lig_v, row_v, dpos_v, tbg_v):
        wid = lax.axis_index("s") * 2 + lax.axis_index("c")
        rbase = wid * _RPW
        ebase = wid * _EPW
        pltpu.sync_copy(eidx_hbm.at[pl.ds(ebase, _EPW)], eidx_v)
        pltpu.sync_copy(res_hbm, res_v)
        pltpu.sync_copy(asym_hbm, asym_v)
        pltpu.sync_copy(lig_hbm, lig_v)

        def row_body(r, carry):
            i_flat = rbase + r
            pltpu.sync_copy(tb_hbm.at[i_flat], row_v)
            bb = (i_flat // _N) * _N
            iq = jnp.full((16,), i_flat, jnp.int32)
            res_q = plsc.load_gather(res_v, [iq])
            asym_q = plsc.load_gather(asym_v, [iq])
            lig_q = plsc.load_gather(lig_v, [iq])
            for c in range(_K // 16):
                off = r * _K + c * 16
                sl = pl.ds(off, 16)
                jj = eidx_v[sl]
                tok = jj + bb
                res_nb = plsc.load_gather(res_v, [tok])
                asym_nb = plsc.load_gather(asym_v, [tok])
                lig_nb = plsc.load_gather(lig_v, [tok])
                tbv = plsc.load_gather(row_v, [jj])
                offs = res_q - res_nb
                t = jnp.clip(offs + 32, 0, 64)
                dfe = jnp.where(asym_q == asym_nb, t, 65).astype(jnp.float32)
                ligm = ((lig_q | lig_nb) > 0).astype(jnp.float32)
                dpos_v[sl] = dfe
                tbg_v[sl] = tbv * ligm
            return carry

        lax.fori_loop(0, _RPW, row_body, 0)
        pltpu.sync_copy(dpos_v, dpos_hbm.at[pl.ds(ebase, _EPW)])
        pltpu.sync_copy(tbg_v, tbg_hbm.at[pl.ds(ebase, _EPW)])

    return k(eidx_flat, tb2, res_flat, asym_flat, lig_flat)


def _topk_body(fa_ref, fq_ref, eidx_ref, dnb_ref):
    xall = fa_ref[0, 0:1, :]
    yall = fa_ref[0, 1:2, :]
    zall = fa_ref[0, 2:3, :]
    xq = fq_ref[0, 0, :, 0:1]
    yq = fq_ref[0, 0, :, 1:2]
    zq = fq_ref[0, 0, :, 2:3]
    dx = xq - xall
    dy = yq - yall
    dz = zq - zall
    d2 = (dx * dx + dy * dy) + dz * dz
    dist = jnp.sqrt(d2 + 1e-6)
    iota = lax.broadcasted_iota(jnp.int32, (1, _N), 1)
    kio = lax.broadcasted_iota(jnp.int32, (1, _K), 1)

    def step(k, carry):
        dcur, eacc, dacc = carry
        m = jnp.min(dcur, axis=1, keepdims=True)
        cand = jnp.where(dcur == m, iota, _N)
        j = jnp.min(cand, axis=1, keepdims=True)
        hitk = kio == k
        eacc = jnp.where(hitk, j, eacc)
        dacc = jnp.where(hitk, m, dacc)
        dcur = jnp.where(iota == j, jnp.inf, dcur)
        return dcur, eacc, dacc

    eacc0 = jnp.zeros((_RB, _K), jnp.int32)
    dacc0 = jnp.zeros((_RB, _K), jnp.float32)
    _, eacc, dacc = lax.fori_loop(0, _K, step, (dist, eacc0, dacc0))
    eidx_ref[0] = eacc
    dnb_ref[0] = dacc


def _tc_topk(feat_all, feat_q):
    return pl.pallas_call(
        _topk_body,
        grid=(_B, _NBLK),
        in_specs=[
            pl.BlockSpec((1, 8, _N), lambda b, blk: (b, 0, 0)),
            pl.BlockSpec((1, 1, _RB, 8), lambda b, blk: (b, blk, 0, 0)),
        ],
        out_specs=(
            pl.BlockSpec((1, _RB, _K), lambda b, blk: (b, blk, 0)),
            pl.BlockSpec((1, _RB, _K), lambda b, blk: (b, blk, 0)),
        ),
        out_shape=(
            jax.ShapeDtypeStruct((_B, _N, _K), jnp.int32),
            jax.ShapeDtypeStruct((_B, _N, _K), jnp.float32),
        ),
    )(feat_all, feat_q)


def _edges_body(aux_ref, wpos_ref, wrbf_ref, wtb_ref, bias_ref, gam_ref,
                bet_ref, out_ref):
    d_col = aux_ref[0, :, 0:1]
    dn_col = aux_ref[0, :, 1:2]
    tb_col = aux_ref[0, :, 2:3]
    iota66 = lax.broadcasted_iota(jnp.int32, (1, 66), 1).astype(jnp.float32)
    oh = (d_col == iota66).astype(jnp.float32)
    mu = 2.0 + (20.0 / 15.0) * lax.broadcasted_iota(
        jnp.int32, (1, 16), 1).astype(jnp.float32)
    z = (dn_col - mu) / 1.25
    rbf = jnp.exp(-(z * z))
    e = (jnp.dot(oh, wpos_ref[...], preferred_element_type=jnp.float32)
         + jnp.dot(rbf, wrbf_ref[...], preferred_element_type=jnp.float32)
         + tb_col * wtb_ref[...]
         + bias_ref[...])
    mean = jnp.sum(e, axis=1, keepdims=True) * (1.0 / 128.0)
    xc = e - mean
    var = jnp.sum(xc * xc, axis=1, keepdims=True) * (1.0 / 128.0)
    out_ref[...] = xc * lax.rsqrt(var + 1e-5) * gam_ref[...] + bet_ref[...]


def _tc_edges(aux, wpos_f, wrbf, wtb, bias, gam, bet):
    return pl.pallas_call(
        _edges_body,
        grid=(_NEB,),
        in_specs=[
            pl.BlockSpec((1, _RE, 8), lambda e: (e, 0, 0)),
            pl.BlockSpec((66, 128), lambda e: (0, 0)),
            pl.BlockSpec((16, 128), lambda e: (0, 0)),
            pl.BlockSpec((1, 128), lambda e: (0, 0)),
            pl.BlockSpec((1, 128), lambda e: (0, 0)),
            pl.BlockSpec((1, 128), lambda e: (0, 0)),
            pl.BlockSpec((1, 128), lambda e: (0, 0)),
        ],
        out_specs=pl.BlockSpec((_RE, 128), lambda e: (e, 0)),
        out_shape=jax.ShapeDtypeStruct((_B * _N * _K, 128), jnp.float32),
    )(aux, wpos_f, wrbf, wtb, bias, gam, bet)


def kernel(coords, token_to_center_atom, token_exists_mask, residue_index,
           asym_id, token_bonds, is_ligand, W_pos, b_pos, W_edge,
           ln_gamma, ln_beta):
    coords_pad = jnp.pad(coords.reshape(_B * _N, 3), ((0, 0), (0, 125)))
    t2c_flat = token_to_center_atom.reshape(_B * _N).astype(jnp.int32)
    xg = _sc_gather_x(coords_pad, t2c_flat)
    x = xg[:, :3].reshape(_B, _N, 3) * token_exists_mask[..., None]

    feat_all = jnp.pad(x.transpose(0, 2, 1), ((0, 0), (0, 5), (0, 0)))
    feat_q = jnp.pad(x.reshape(_B, _NBLK, _RB, 3),
                     ((0, 0), (0, 0), (0, 0), (0, 5)))
    e_idx, d_nb = _tc_topk(feat_all, feat_q)

    dpos, tbg = _sc_edges(
        e_idx.reshape(-1),
        token_bonds.reshape(_B * _N, _N),
        residue_index.reshape(-1).astype(jnp.int32),
        asym_id.reshape(-1).astype(jnp.int32),
        is_ligand.reshape(-1).astype(jnp.int32),
    )

    wpos_f = jnp.dot(W_pos, W_edge[:16], preferred_element_type=jnp.float32)
    bias = jnp.dot(b_pos[None, :], W_edge[:16],
                   preferred_element_type=jnp.float32)
    aux = jnp.stack([dpos, d_nb.reshape(-1), tbg], axis=-1)
    aux = jnp.pad(aux, ((0, 0), (0, 5))).reshape(_NEB, _RE, 8)
    e_out = _tc_edges(aux, wpos_f, W_edge[16:32], W_edge[32:33], bias,
                      ln_gamma[None, :], ln_beta[None, :])

    return (e_out.reshape(_B, _N, _K, 128), e_idx, d_nb)


# speedup vs baseline: 2.0302x; 1.0107x over previous
import functools

import jax
import jax.numpy as jnp
from jax import lax
from jax.experimental import pallas as pl
from jax.experimental.pallas import tpu as pltpu
from jax.experimental.pallas import tpu_sc as plsc

_B = 4
_N = 2048
_K = 32
_NW = 32
_TPW = _B * _N // _NW
_RPW = _TPW
_EPW = _RPW * _K

_RB = 256
_NBLK = _N // _RB

_RE = 2048
_NEB = _B * _N * _K // _RE


def _sc_gather_x(coords_pad, t2c_flat):
    mesh = plsc.VectorSubcoreMesh(core_axis_name="c", subcore_axis_name="s")

    @functools.partial(
        pl.kernel,
        mesh=mesh,
        out_type=jax.ShapeDtypeStruct((_B * _N, 128), jnp.float32),
        scratch_types=[
            pltpu.VMEM((_TPW,), jnp.int32),
            pltpu.VMEM((_TPW, 128), jnp.float32),
            pltpu.SemaphoreType.DMA,
        ],
    )
    def k(coords_hbm, idx_hbm, out_hbm, idx_v, rows_v, sem):
        wid = lax.axis_index("s") * 2 + lax.axis_index("c")
        base = wid * _TPW
        pltpu.sync_copy(idx_hbm.at[pl.ds(base, _TPW)], idx_v)
        bb = (base // _N) * _N
        for c in range(_TPW // 16):
            sl = pl.ds(c * 16, 16)
            idx_v[sl] = idx_v[sl] + bb
        pltpu.async_copy(coords_hbm.at[idx_v], rows_v, sem).wait()
        pltpu.sync_copy(rows_v, out_hbm.at[pl.ds(base, _TPW)])

    return k(coords_pad, t2c_flat)


def _sc_edges(eidx_flat, tb2, res_flat, asym_flat, lig_flat):
    mesh = plsc.VectorSubcoreMesh(core_axis_name="c", subcore_axis_name="s")
    nedges = _B * _N * _K

    @functools.partial(
        pl.kernel,
        mesh=mesh,
        out_type=(
            jax.ShapeDtypeStruct((nedges,), jnp.float32),
            jax.ShapeDtypeStruct((nedges,), jnp.float32),
        ),
        compiler_params=pltpu.CompilerParams(needs_layout_passes=False),
        scratch_types=[
            pltpu.VMEM((_EPW,), jnp.int32),
            pltpu.VMEM((_B * _N,), jnp.int32),
            pltpu.VMEM((_B * _N,), jnp.int32),
            pltpu.VMEM((_B * _N,), jnp.int32),
            pltpu.VMEM((_N,), jnp.float32),
            pltpu.VMEM((_EPW,), jnp.float32),
            pltpu.VMEM((_EPW,), jnp.float32),
        ],
    )
    def k(eidx_hbm, tb_hbm, res_hbm, asym_hbm, lig_hbm, dpos_hbm, tbg_hbm,
          eidx_v, res_v, asym_v, lig_v, row_v, dpos_v, tbg_v):
        wid = lax.axis_index("s") * 2 + lax.axis_index("c")
        rbase = wid * _RPW
        ebase = wid * _EPW
        pltpu.sync_copy(eidx_hbm.at[pl.ds(ebase, _EPW)], eidx_v)
        pltpu.sync_copy(res_hbm, res_v)
        pltpu.sync_copy(asym_hbm, asym_v)
        pltpu.sync_copy(lig_hbm, lig_v)

        def row_body(r, carry):
            i_flat = rbase + r
            pltpu.sync_copy(tb_hbm.at[i_flat], row_v)
            bb = (i_flat // _N) * _N
            iq = jnp.full((16,), i_flat, jnp.int32)
            res_q = plsc.load_gather(res_v, [iq])
            asym_q = plsc.load_gather(asym_v, [iq])
            lig_q = plsc.load_gather(lig_v, [iq])
            for c in range(_K // 16):
                off = r * _K + c * 16
                sl = pl.ds(off, 16)
                jj = eidx_v[sl]
                tok = jj + bb
                res_nb = plsc.load_gather(res_v, [tok])
                asym_nb = plsc.load_gather(asym_v, [tok])
                lig_nb = plsc.load_gather(lig_v, [tok])
                tbv = plsc.load_gather(row_v, [jj])
                offs = res_q - res_nb
                t = jnp.clip(offs + 32, 0, 64)
                dfe = jnp.where(asym_q == asym_nb, t, 65).astype(jnp.float32)
                ligm = ((lig_q | lig_nb) > 0).astype(jnp.float32)
                dpos_v[sl] = dfe
                tbg_v[sl] = tbv * ligm
            return carry

        lax.fori_loop(0, _RPW, row_body, 0)
        pltpu.sync_copy(dpos_v, dpos_hbm.at[pl.ds(ebase, _EPW)])
        pltpu.sync_copy(tbg_v, tbg_hbm.at[pl.ds(ebase, _EPW)])

    return k(eidx_flat, tb2, res_flat, asym_flat, lig_flat)


def _topk_body(fa_ref, fq_ref, eidx_ref, dnb_ref):
    xall = fa_ref[0, 0:1, :]
    yall = fa_ref[0, 1:2, :]
    zall = fa_ref[0, 2:3, :]
    xq = fq_ref[0, 0, :, 0:1]
    yq = fq_ref[0, 0, :, 1:2]
    zq = fq_ref[0, 0, :, 2:3]
    dx = xq - xall
    dy = yq - yall
    dz = zq - zall
    d2 = (dx * dx + dy * dy) + dz * dz
    iota = lax.broadcasted_iota(jnp.int32, (1, _N), 1)
    kio = lax.broadcasted_iota(jnp.int32, (1, _K), 1)
    m0 = jnp.min(d2, axis=1, keepdims=True)

    def step(k, carry):
        dcur, m, eacc, dacc = carry
        cand = jnp.where(dcur == m, iota, _N)
        j = jnp.min(cand, axis=1, keepdims=True)
        hitk = kio == k
        eacc = jnp.where(hitk, j, eacc)
        dacc = jnp.where(hitk, m, dacc)
        dcur = jnp.where(iota == j, jnp.inf, dcur)
        m = jnp.min(dcur, axis=1, keepdims=True)
        return dcur, m, eacc, dacc

    eacc0 = jnp.zeros((_RB, _K), jnp.int32)
    dacc0 = jnp.zeros((_RB, _K), jnp.float32)
    _, _, eacc, dacc = lax.fori_loop(0, _K, step, (d2, m0, eacc0, dacc0))
    eidx_ref[0] = eacc
    dnb_ref[0] = jnp.sqrt(dacc + 1e-6)


def _tc_topk(feat_all, feat_q):
    return pl.pallas_call(
        _topk_body,
        grid=(_B, _NBLK),
        in_specs=[
            pl.BlockSpec((1, 8, _N), lambda b, blk: (b, 0, 0)),
            pl.BlockSpec((1, 1, _RB, 8), lambda b, blk: (b, blk, 0, 0)),
        ],
        out_specs=(
            pl.BlockSpec((1, _RB, _K), lambda b, blk: (b, blk, 0)),
            pl.BlockSpec((1, _RB, _K), lambda b, blk: (b, blk, 0)),
        ),
        out_shape=(
            jax.ShapeDtypeStruct((_B, _N, _K), jnp.int32),
            jax.ShapeDtypeStruct((_B, _N, _K), jnp.float32),
        ),
    )(feat_all, feat_q)


def _edges_body(aux_ref, wpos_ref, wrbf_ref, wtb_ref, bias_ref, gam_ref,
                bet_ref, out_ref):
    d_col = aux_ref[0, :, 0:1]
    dn_col = aux_ref[0, :, 1:2]
    tb_col = aux_ref[0, :, 2:3]
    iota66 = lax.broadcasted_iota(jnp.int32, (1, 66), 1).astype(jnp.float32)
    oh = (d_col == iota66).astype(jnp.float32)
    mu = 2.0 + (20.0 / 15.0) * lax.broadcasted_iota(
        jnp.int32, (1, 16), 1).astype(jnp.float32)
    z = (dn_col - mu) / 1.25
    rbf = jnp.exp(-(z * z))
    e = (jnp.dot(oh, wpos_ref[...], preferred_element_type=jnp.float32)
         + jnp.dot(rbf, wrbf_ref[...], preferred_element_type=jnp.float32)
         + tb_col * wtb_ref[...]
         + bias_ref[...])
    mean = jnp.sum(e, axis=1, keepdims=True) * (1.0 / 128.0)
    xc = e - mean
    var = jnp.sum(xc * xc, axis=1, keepdims=True) * (1.0 / 128.0)
    out_ref[...] = xc * lax.rsqrt(var + 1e-5) * gam_ref[...] + bet_ref[...]


def _tc_edges(aux, wpos_f, wrbf, wtb, bias, gam, bet):
    return pl.pallas_call(
        _edges_body,
        grid=(_NEB,),
        in_specs=[
            pl.BlockSpec((1, _RE, 8), lambda e: (e, 0, 0)),
            pl.BlockSpec((66, 128), lambda e: (0, 0)),
            pl.BlockSpec((16, 128), lambda e: (0, 0)),
            pl.BlockSpec((1, 128), lambda e: (0, 0)),
            pl.BlockSpec((1, 128), lambda e: (0, 0)),
            pl.BlockSpec((1, 128), lambda e: (0, 0)),
            pl.BlockSpec((1, 128), lambda e: (0, 0)),
        ],
        out_specs=pl.BlockSpec((_RE, 128), lambda e: (e, 0)),
        out_shape=jax.ShapeDtypeStruct((_B * _N * _K, 128), jnp.float32),
    )(aux, wpos_f, wrbf, wtb, bias, gam, bet)


def kernel(coords, token_to_center_atom, token_exists_mask, residue_index,
           asym_id, token_bonds, is_ligand, W_pos, b_pos, W_edge,
           ln_gamma, ln_beta):
    coords_pad = jnp.pad(coords.reshape(_B * _N, 3), ((0, 0), (0, 125)))
    t2c_flat = token_to_center_atom.reshape(_B * _N).astype(jnp.int32)
    xg = _sc_gather_x(coords_pad, t2c_flat)
    x = xg[:, :3].reshape(_B, _N, 3) * token_exists_mask[..., None]

    feat_all = jnp.pad(x.transpose(0, 2, 1), ((0, 0), (0, 5), (0, 0)))
    feat_q = jnp.pad(x.reshape(_B, _NBLK, _RB, 3),
                     ((0, 0), (0, 0), (0, 0), (0, 5)))
    e_idx, d_nb = _tc_topk(feat_all, feat_q)

    dpos, tbg = _sc_edges(
        e_idx.reshape(-1),
        token_bonds.reshape(_B * _N, _N),
        residue_index.reshape(-1).astype(jnp.int32),
        asym_id.reshape(-1).astype(jnp.int32),
        is_ligand.reshape(-1).astype(jnp.int32),
    )

    wpos_f = jnp.dot(W_pos, W_edge[:16], preferred_element_type=jnp.float32)
    bias = jnp.dot(b_pos[None, :], W_edge[:16],
                   preferred_element_type=jnp.float32)
    aux = jnp.stack([dpos, d_nb.reshape(-1), tbg], axis=-1)
    aux = jnp.pad(aux, ((0, 0), (0, 5))).reshape(_NEB, _RE, 8)
    e_out = _tc_edges(aux, wpos_f, W_edge[16:32], W_edge[32:33], bias,
                      ln_gamma[None, :], ln_beta[None, :])

    return (e_out.reshape(_B, _N, _K, 128), e_idx, d_nb)
